# Initial kernel scaffold; baseline (speedup 1.0000x reference)
import jax
import jax.numpy as jnp
from jax.experimental import pallas as pl


def kernel(op_feats, device_feats, link_efeats, link_src, link_dst, prev_efeats, prev_src, prev_dst, succ_efeats, succ_src, succ_dst, place_efeats, place_src, place_dst, serve_efeats, serve_src, serve_dst, params):
    raise NotImplementedError("write your pallas kernel here")


# speedup vs baseline: 2.8664x; 2.8664x over previous
import functools
from typing import Any

import jax
import jax.numpy as jnp
from jax import lax
from jax.experimental import pallas as pl
from jax.experimental.pallas import tpu as pltpu
from jax.experimental.pallas import tpu_sc as plsc

N_OP = 50000
N_DEV = 1000
D = 128
DE = 16
H = 2
F = 128
HF = H * F
EPS = 1e-3

NC = 2
NS = 16
NW = NC * NS
LANES = 16

EB = 128
EBD = 64

NP_OP = 50048
NP_DEV = 1024
ZR_OP = 3128
ZR_DEV = 64

_ETS = (
    ("link", "dev", "dev", N_DEV, N_DEV, 16000),
    ("prev", "op", "op", N_OP, N_OP, 160000),
    ("succ", "op", "op", N_OP, N_OP, 160000),
    ("place", "op", "dev", N_OP, N_DEV, 50000),
    ("serve", "dev", "op", N_DEV, N_OP, 50000),
)


def _pad_edges(e):
  return ((e + 4095) // 4096) * 4096


_EPAD = {name: _pad_edges(e) for name, *_, e in _ETS}


def _nodeproj_body(x_ref, w_ref, b_ref, *out_refs):
  y = jnp.dot(x_ref[...], w_ref[...], preferred_element_type=jnp.float32)
  y = y + b_ref[...]
  n_a = len(out_refs) - 1
  for i in range(n_a):
    out_refs[i][...] = y[:, i * HF:(i + 1) * HF]
  out_refs[n_a][...] = y[:, n_a * HF:n_a * HF + 16]


def _node_proj(x, w, b, n_a, blk):
  n = x.shape[0]
  c = w.shape[1]
  grid = (n // blk,)
  outs = [jax.ShapeDtypeStruct((n, HF), jnp.float32) for _ in range(n_a)]
  outs.append(jax.ShapeDtypeStruct((n, 16), jnp.float32))
  out_specs = [pl.BlockSpec((blk, HF), lambda i: (i, 0)) for _ in range(n_a)]
  out_specs.append(pl.BlockSpec((blk, 16), lambda i: (i, 0)))
  return pl.pallas_call(
      _nodeproj_body,
      grid=grid,
      in_specs=[
          pl.BlockSpec((blk, D), lambda i: (i, 0)),
          pl.BlockSpec((D, c), lambda i: (0, 0)),
          pl.BlockSpec((1, c), lambda i: (0, 0)),
      ],
      out_specs=out_specs,
      out_shape=outs,
  )(x, w, b)


def _edgeproj_body(e_ref, w_ref, b_ref, o_ref):
  o_ref[...] = (
      jnp.dot(e_ref[...], w_ref[...], preferred_element_type=jnp.float32)
      + b_ref[...]
  )


def _edge_proj(ef, w, b, blk):
  e = ef.shape[0]
  c = w.shape[1]
  return pl.pallas_call(
      _edgeproj_body,
      grid=(e // blk,),
      in_specs=[
          pl.BlockSpec((blk, DE), lambda i: (i, 0)),
          pl.BlockSpec((DE, c), lambda i: (0, 0)),
          pl.BlockSpec((1, c), lambda i: (0, 0)),
      ],
      out_specs=pl.BlockSpec((blk, c), lambda i: (i, 0)),
      out_shape=jax.ShapeDtypeStruct((e, c), jnp.float32),
  )(ef, w, b)


def _messages_body(blk, nb, ga_ref, e_ref, w_ref, b_ref, a_ref, lt_ref,
                   o_ref, o2_ref, carry, carry2):
  b = pl.program_id(0)

  @pl.when(b == 0)
  def _():
    carry[...] = jnp.zeros_like(carry)
    carry2[...] = jnp.zeros_like(carry2)

  @pl.when(b < nb)
  def _():
    y = jnp.dot(e_ref[...], w_ref[...], preferred_element_type=jnp.float32)
    y = y + b_ref[...] + ga_ref[...]
    ee = a_ref[...]
    a0 = ee[:, 0:1]
    a1 = ee[:, 1:2]
    scale = jnp.concatenate(
        [jnp.broadcast_to(a0, (a0.shape[0], F)),
         jnp.broadcast_to(a1, (a1.shape[0], F))], axis=1)
    m = y * scale
    lt = lt_ref[...]
    cs = jnp.dot(lt, m, preferred_element_type=jnp.float32)
    cs = cs + carry[0:1, :]
    o_ref[...] = cs
    carry[...] = jnp.broadcast_to(cs[blk - 1:blk, :], carry.shape)
    eep = jnp.concatenate([ee, jnp.zeros((blk, F - 2), jnp.float32)], axis=1)
    cs2 = jnp.dot(lt, eep, preferred_element_type=jnp.float32)
    cs2 = cs2 + carry2[0:1, :]
    o2_ref[...] = cs2
    carry2[...] = jnp.broadcast_to(cs2[blk - 1:blk, :], carry2.shape)

  @pl.when(b >= nb)
  def _():
    o_ref[...] = jnp.zeros_like(o_ref)
    o2_ref[...] = jnp.zeros_like(o2_ref)


def _messages(ga, ef, w, b, alpha, lt, blk):
  e = ef.shape[0]
  nb = e // blk
  clamp = lambda i: jnp.minimum(i, nb - 1)
  return pl.pallas_call(
      functools.partial(_messages_body, blk, nb),
      grid=(nb + 1,),
      in_specs=[
          pl.BlockSpec((blk, HF), lambda i: (clamp(i), 0)),
          pl.BlockSpec((blk, DE), lambda i: (clamp(i), 0)),
          pl.BlockSpec((DE, HF), lambda i: (0, 0)),
          pl.BlockSpec((1, HF), lambda i: (0, 0)),
          pl.BlockSpec((blk, 2), lambda i: (clamp(i), 0)),
          pl.BlockSpec((blk, blk), lambda i: (0, 0)),
      ],
      out_specs=[pl.BlockSpec((blk, HF), lambda i: (i, 0)),
                 pl.BlockSpec((blk, F), lambda i: (i, 0))],
      out_shape=[jax.ShapeDtypeStruct((e + blk, HF), jnp.float32),
                 jax.ShapeDtypeStruct((e + blk, F), jnp.float32)],
      scratch_shapes=[pltpu.VMEM((8, HF), jnp.float32),
                      pltpu.VMEM((8, F), jnp.float32)],
  )(ga, ef, w, b, alpha, lt)


def _final_body(scale, n_et, *refs):
  ins = refs[:4 * n_et]
  x_ref, w_ref, b_ref, o_ref = refs[4 * n_et:]
  acc = None
  for k in range(n_et):
    num = ins[4 * k][...] - ins[4 * k + 1][...]
    den = ins[4 * k + 2][...] - ins[4 * k + 3][...]
    d0 = den[:, 0:1]
    d1 = den[:, 1:2]
    blk = num.shape[0]
    inv0 = jnp.where(d0 > 0, 1.0 / d0, 0.0)
    inv1 = jnp.where(d1 > 0, 1.0 / d1, 0.0)
    inv = jnp.concatenate([jnp.broadcast_to(inv0, (blk, F)),
                           jnp.broadcast_to(inv1, (blk, F))], axis=1)
    d = num * inv
    acc = d if acc is None else acc + d
  y = jnp.dot(acc * scale, w_ref[...], preferred_element_type=jnp.float32)
  o_ref[...] = y + b_ref[...] + x_ref[...]


def _final(gs, x, w, b, scale, blk):
  n = x.shape[0]
  n_et = len(gs) // 4
  widths = [HF, HF, F, F] * n_et
  specs = [pl.BlockSpec((blk, wd), lambda i: (i, 0)) for wd in widths]
  specs += [
      pl.BlockSpec((blk, D), lambda i: (i, 0)),
      pl.BlockSpec((HF, D), lambda i: (0, 0)),
      pl.BlockSpec((1, D), lambda i: (0, 0)),
  ]
  return pl.pallas_call(
      functools.partial(_final_body, scale, n_et),
      grid=(n // blk,),
      in_specs=specs,
      out_specs=pl.BlockSpec((blk, D), lambda i: (i, 0)),
      out_shape=jax.ShapeDtypeStruct((n, D), jnp.float32),
  )(*gs, x, w, b)


_MESH = plsc.VectorSubcoreMesh(core_axis_name="c", subcore_axis_name="s",
                               num_cores=NC, num_subcores=NS)


def _wid():
  return lax.axis_index("s") * NC + lax.axis_index("c")


def _softmax_kernel(ets):
  n_et = len(ets)

  def body(*refs):
    i = 0
    ins = []
    for _ in range(n_et):
      ins.append(refs[i:i + 8])
      i += 8
    outs = []
    for _ in range(n_et):
      outs.append(refs[i:i + 4])
      i += 4
    dsp = []
    for _ in range(n_et):
      dsp.append(refs[i:i + 2])
      i += 2
    (zbuf, sidx, didx, ga0, ga1, ge0, ge1, gb0, gb1, ee0, ee1, dn0, dn1,
     al0, al1) = refs[i:]

    cid = lax.axis_index("c")
    tid = lax.axis_index("s")
    wid = _wid()

    def zfill(r, _):
      zbuf[pl.ds(r * LANES, LANES)] = jnp.zeros((LANES,), jnp.float32)
      return ()
    lax.fori_loop(0, ZR_OP // LANES, zfill, (), unroll=4)

    for k in range(n_et):
      npd = ets[k]["np_dst"]
      rows = npd // NS
      pltpu.sync_copy(zbuf.at[pl.ds(0, rows)],
                      dsp[k][0].at[pl.ds(tid * rows, rows)])
      pltpu.sync_copy(zbuf.at[pl.ds(0, rows)],
                      dsp[k][1].at[pl.ds(tid * rows, rows)])
    plsc.subcore_barrier()

    for k in range(n_et):
      src_r, dst_r, ela0_r, ela1_r, elb0_r, elb1_r, er0_r, er1_r = ins[k]
      ee0_r, ee1_r, _, _ = outs[k]
      ep = ets[k]["ep"]
      e_real = ets[k]["e"]
      per = ep // NS
      nblk = per // EB

      def p1(b, _, src_r=src_r, dst_r=dst_r, ela0_r=ela0_r, ela1_r=ela1_r,
             elb0_r=elb0_r, elb1_r=elb1_r, er0_r=er0_r, er1_r=er1_r,
             ee0_r=ee0_r, ee1_r=ee1_r, k=k, per=per, e_real=e_real):
        base = tid * per + b * EB
        pltpu.sync_copy(src_r.at[pl.ds(base, EB)], sidx)
        pltpu.sync_copy(dst_r.at[pl.ds(base, EB)], didx)
        pltpu.sync_copy(ela0_r.at[sidx], ga0)
        pltpu.sync_copy(ela1_r.at[sidx], ga1)
        pltpu.sync_copy(er0_r.at[didx], ge0)
        pltpu.sync_copy(er1_r.at[didx], ge1)
        pltpu.sync_copy(elb0_r.at[pl.ds(base, EB)], gb0)
        pltpu.sync_copy(elb1_r.at[pl.ds(base, EB)], gb1)
        for j in range(EB // LANES):
          sl = pl.ds(j * LANES, LANES)
          idxv = lax.iota(jnp.int32, LANES) + (base + j * LANES)
          valid = idxv < e_real
          for (gar, ger, gbr, eer) in ((ga0, ge0, gb0, ee0),
                                       (ga1, ge1, gb1, ee1)):
            ev = gar[sl] + ger[sl] + gbr[sl]
            ev = jnp.where(ev >= 0, ev, 0.2 * ev)
            ev = jnp.where(valid, jnp.exp(ev), 0.0)
            eer[sl] = ev
        pltpu.sync_copy(ee0, ee0_r.at[pl.ds(base, EB)])
        pltpu.sync_copy(ee1, ee1_r.at[pl.ds(base, EB)])
        pltpu.sync_copy(ee0, dsp[k][0].at[didx], add=True)
        pltpu.sync_copy(ee1, dsp[k][1].at[didx], add=True)
        return ()

      lax.fori_loop(0, nblk, p1, ())

    plsc.subcore_barrier()

    for k in range(n_et):
      dst_r = ins[k][1]
      ee0_r, ee1_r, al0_r, al1_r = outs[k]
      ep = ets[k]["ep"]
      per = ep // NW
      nblk = per // EB

      def p2(b, _, dst_r=dst_r, ee0_r=ee0_r, ee1_r=ee1_r, al0_r=al0_r,
             al1_r=al1_r, k=k, per=per):
        base = wid * per + b * EB
        pltpu.sync_copy(dst_r.at[pl.ds(base, EB)], didx)
        pltpu.sync_copy(dsp[k][0].at[didx], dn0)
        pltpu.sync_copy(dsp[k][1].at[didx], dn1)
        pltpu.sync_copy(ee0_r.at[pl.ds(base, EB)], ee0)
        pltpu.sync_copy(ee1_r.at[pl.ds(base, EB)], ee1)
        for j in range(EB // LANES):
          sl = pl.ds(j * LANES, LANES)
          for (eer, dnr, alr) in ((ee0, dn0, al0), (ee1, dn1, al1)):
            dv = dnr[sl]
            av = jnp.where(dv > 0, eer[sl] / dv, 0.0)
            alr[sl] = av
        pltpu.sync_copy(al0, al0_r.at[pl.ds(base, EB)])
        pltpu.sync_copy(al1, al1_r.at[pl.ds(base, EB)])
        return ()

      lax.fori_loop(0, nblk, p2, ())

  out_type = []
  for et in ets:
    out_type.extend([jax.ShapeDtypeStruct((et["ep"],), jnp.float32)] * 4)
  scratch = []
  for et in ets:
    scratch.extend(
        [pltpu.VMEM_SHARED((et["np_dst"],), jnp.float32)] * 2)
  scratch.extend([
      pltpu.VMEM((ZR_OP,), jnp.float32),
      pltpu.VMEM((EB,), jnp.int32),
      pltpu.VMEM((EB,), jnp.int32),
      pltpu.VMEM((EB,), jnp.float32),
      pltpu.VMEM((EB,), jnp.float32),
      pltpu.VMEM((EB,), jnp.float32),
      pltpu.VMEM((EB,), jnp.float32),
      pltpu.VMEM((EB,), jnp.float32),
      pltpu.VMEM((EB,), jnp.float32),
      pltpu.VMEM((EB,), jnp.float32),
      pltpu.VMEM((EB,), jnp.float32),
      pltpu.VMEM((EB,), jnp.float32),
      pltpu.VMEM((EB,), jnp.float32),
      pltpu.VMEM((EB,), jnp.float32),
      pltpu.VMEM((EB,), jnp.float32),
  ])
  return pl.kernel(body, out_type=tuple(out_type), mesh=_MESH,
                   scratch_types=tuple(scratch))


def _rowgather_kernel(jobs):
  n_j = len(jobs)

  def body(*refs):
    ins = [refs[2 * k:2 * k + 2] for k in range(n_j)]
    outs = refs[2 * n_j:3 * n_j]
    sidx, b16, b128, b256 = refs[3 * n_j:]
    cid = lax.axis_index("c")
    tid = lax.axis_index("s")
    wid = (cid * NS + tid)

    for k in range(n_j):
      idx_r, tab_r = ins[k]
      out_r = outs[k]
      ni = jobs[k]["ni"]
      w = jobs[k]["w"]
      buf = b16 if w == DE else (b128 if w == F else b256)
      rows = ni // NW
      nfull = rows // EB
      rem = rows - nfull * EB
      base0 = wid * rows

      def blk(b, _, idx_r=idx_r, tab_r=tab_r, out_r=out_r, buf=buf,
              base0=base0):
        base = base0 + b * EB
        pltpu.sync_copy(idx_r.at[pl.ds(base, EB)], sidx)
        pltpu.sync_copy(tab_r.at[sidx], buf)
        pltpu.sync_copy(buf, out_r.at[pl.ds(base, EB), :])
        return ()

      lax.fori_loop(0, nfull, blk, ())
      if rem:
        base = base0 + nfull * EB
        pltpu.sync_copy(idx_r.at[pl.ds(base, rem)], sidx.at[pl.ds(0, rem)])
        pltpu.sync_copy(tab_r.at[sidx.at[pl.ds(0, rem)]],
                        buf.at[pl.ds(0, rem), :])
        pltpu.sync_copy(buf.at[pl.ds(0, rem), :],
                        out_r.at[pl.ds(base, rem), :])

  out_type = tuple(
      jax.ShapeDtypeStruct((j["ni"], j["w"]), jnp.float32) for j in jobs)
  scratch = (
      pltpu.VMEM((EB,), jnp.int32),
      pltpu.VMEM((EB, DE), jnp.float32),
      pltpu.VMEM((EB, F), jnp.float32),
      pltpu.VMEM((EB, HF), jnp.float32),
  )
  return pl.kernel(body, out_type=out_type, mesh=_MESH,
                   scratch_types=scratch)


def kernel(op_feats, device_feats,
           link_efeats, link_src, link_dst,
           prev_efeats, prev_src, prev_dst,
           succ_efeats, succ_src, succ_dst,
           place_efeats, place_src, place_dst,
           serve_efeats, serve_src, serve_dst,
           params: dict[str, Any]):
  tables = {"op": op_feats, "dev": device_feats}
  raw = {
      "link": (link_efeats, link_src, link_dst),
      "prev": (prev_efeats, prev_src, prev_dst),
      "succ": (succ_efeats, succ_src, succ_dst),
      "place": (place_efeats, place_src, place_dst),
      "serve": (serve_efeats, serve_src, serve_dst),
  }
  c = 1.0 / jnp.sqrt(1.0 + EPS)

  folded = {}
  for name, st, dt, n_src, n_dst, e in _ETS:
    p = params[name]
    gs = p["bn_src_gamma"] * c
    gd = p["bn_dst_gamma"] * c
    w1f = p["W_src"][:D] * gs[None, :]
    w2f = p["W_src"][D:] * gs[None, :]
    bs = p["bn_src_beta"]
    al = p["attn_l"].reshape(H, F)
    ar = p["attn_r"].reshape(H, F)
    mal = jnp.zeros((HF, H), jnp.float32)
    mar = jnp.zeros((HF, H), jnp.float32)
    for h in range(H):
      mal = mal.at[h * F:(h + 1) * F, h].set(al[h])
      mar = mar.at[h * F:(h + 1) * F, h].set(ar[h])
    wdf = p["W_dst"] * gd[None, :]
    folded[name] = dict(
        w1f=w1f, w2f=w2f, bs=bs,
        ma=w1f @ mal,
        mb=w2f @ mal, cb=bs @ mal,
        mr=wdf @ mar, cr=(p["bn_dst_beta"] @ mar),
        st=st, dt=dt, n_src=n_src, n_dst=n_dst, e=e,
    )

  npg = {"op": 51200, "dev": 1024}
  ep_d = {}
  meta = {}
  for name in raw:
    ef, s_i, d_i = raw[name]
    f = folded[name]
    e = f["e"]
    epad = _EPAD[name]
    pad = epad - e
    efp = jnp.pad(ef, ((0, pad), (0, 0)))
    sp = jnp.pad(s_i, (0, pad))
    d0 = jnp.where(jnp.arange(epad) < e, jnp.pad(d_i, (0, pad)),
                   f["n_dst"])
    perm = jnp.argsort(d0, stable=True).astype(jnp.int32)
    src_s = jnp.take(sp, perm)
    dst_s = jnp.take(d0, perm)
    dst_g = jnp.minimum(dst_s, f["n_dst"] - 1)
    ng = npg[f["dt"]]
    ptr = jnp.searchsorted(dst_s, jnp.arange(ng + 1, dtype=jnp.int32),
                           side="left").astype(jnp.int32)
    idx_a = jnp.where(ptr[1:] - 1 >= 0, ptr[1:] - 1, epad)
    idx_b = jnp.where(ptr[:-1] - 1 >= 0, ptr[:-1] - 1, epad)
    ep_d[name] = (efp, src_s, dst_g, perm)
    meta[name] = (idx_a.astype(jnp.int32), idx_b.astype(jnp.int32))

  w_op = jnp.concatenate(
      [folded["prev"]["w1f"], folded["succ"]["w1f"], folded["place"]["w1f"],
       folded["prev"]["ma"], folded["succ"]["ma"], folded["place"]["ma"],
       folded["prev"]["mr"], folded["succ"]["mr"], folded["serve"]["mr"],
       jnp.zeros((D, 4), jnp.float32)], axis=1)
  b_op = jnp.concatenate(
      [jnp.zeros((3 * HF + 6,), jnp.float32),
       folded["prev"]["cr"], folded["succ"]["cr"], folded["serve"]["cr"],
       jnp.zeros((4,), jnp.float32)])[None, :]
  a_prev, a_succ, a_place, small_op = _node_proj(op_feats, w_op, b_op, 3, 1000)

  w_dev = jnp.concatenate(
      [folded["link"]["w1f"], folded["serve"]["w1f"],
       folded["link"]["ma"], folded["serve"]["ma"],
       folded["link"]["mr"], folded["place"]["mr"],
       jnp.zeros((D, 8), jnp.float32)], axis=1)
  b_dev = jnp.concatenate(
      [jnp.zeros((2 * HF + 4,), jnp.float32),
       folded["link"]["cr"], folded["place"]["cr"],
       jnp.zeros((8,), jnp.float32)])[None, :]
  a_link, a_serve, small_dev = _node_proj(device_feats, w_dev, b_dev, 2, 1000)

  a_tab = {"prev": a_prev, "succ": a_succ, "place": a_place,
           "link": a_link, "serve": a_serve}
  ela = {"prev": small_op[:, 0:2], "succ": small_op[:, 2:4],
         "place": small_op[:, 4:6], "link": small_dev[:, 0:2],
         "serve": small_dev[:, 2:4]}
  er = {"prev": small_op[:, 6:8], "succ": small_op[:, 8:10],
        "serve": small_op[:, 10:12], "link": small_dev[:, 4:6],
        "place": small_dev[:, 6:8]}

  order = ["link", "prev", "succ", "place", "serve"]

  g1 = _rowgather_kernel([dict(ni=_EPAD[n], w=HF) for n in order])
  g1_in = []
  for n in order:
    g1_in.extend([ep_d[n][1], a_tab[n]])
  g1_out = g1(*g1_in)
  ga = dict(zip(order, g1_out))
  ef_s = {n: jnp.take(ep_d[n][0], ep_d[n][3], axis=0) for n in order}

  elb = {}
  for n in order:
    f = folded[n]
    w = jnp.concatenate([f["mb"], jnp.zeros((DE, 14), jnp.float32)], axis=1)
    b = jnp.concatenate([f["cb"], jnp.zeros((14,), jnp.float32)])[None, :]
    y = _edge_proj(ef_s[n], w, b, 4096)
    elb[n] = (jnp.copy(y[:, 0]), jnp.copy(y[:, 1]))

  np_dst = {"op": NP_OP, "dev": NP_DEV}
  ets_meta = [dict(ep=_EPAD[n], e=folded[n]["e"],
                   np_dst=np_dst[folded[n]["dt"]]) for n in order]
  sm = _softmax_kernel(ets_meta)
  sm_in = []
  for n in order:
    efp, src_s, dst_g, _ = ep_d[n]
    sm_in.extend([
        src_s, dst_g,
        jnp.copy(ela[n][:, 0]), jnp.copy(ela[n][:, 1]),
        elb[n][0], elb[n][1],
        jnp.copy(er[n][:, 0]), jnp.copy(er[n][:, 1]),
    ])
  sm_out = sm(*sm_in)
  eew = {}
  for i, n in enumerate(order):
    ee0, ee1, al0, al1 = sm_out[4 * i:4 * i + 4]
    eew[n] = (ee0, ee1)

  lt = jnp.tril(jnp.ones((512, 512), jnp.float32))
  cs = {}
  cse = {}
  for n in order:
    f = folded[n]
    ee2 = jnp.stack(eew[n], axis=1)
    cs[n], cse[n] = _messages(ga[n], ef_s[n], f["w2f"], f["bs"][None, :],
                              ee2, lt, 512)

  jobs = []
  for n in order:
    ng = npg[folded[n]["dt"]]
    jobs.extend([dict(ni=ng, w=HF), dict(ni=ng, w=HF),
                 dict(ni=ng, w=F), dict(ni=ng, w=F)])
  g3 = _rowgather_kernel(jobs)
  g3_in = []
  for n in order:
    ia, ib = meta[n]
    g3_in.extend([ia, cs[n], ib, cs[n], ia, cse[n], ib, cse[n]])
  g3_out = g3(*g3_in)
  gpre = {}
  for i, n in enumerate(order):
    gpre[n] = g3_out[4 * i:4 * i + 4]

  op_gs = []
  for n in ("prev", "succ", "serve"):
    op_gs.extend(gpre[n])
  dev_gs = []
  for n in ("link", "place"):
    dev_gs.extend(gpre[n])
  out_op = _final(op_gs, op_feats, params["op_fc_W"],
                  params["op_fc_b"][None, :], 1.0 / 3.0, 1000)
  out_dev = _final(dev_gs, device_feats, params["device_fc_W"],
                   params["device_fc_b"][None, :], 1.0 / 2.0, 1000)
  return (out_op, out_dev)
